# speedup vs baseline: 11.7151x; 11.7151x over previous
import functools

import jax
import jax.numpy as jnp
from jax import lax
from jax.experimental import pallas as pl
from jax.experimental.pallas import tpu as pltpu
from jax.experimental.pallas import tpu_sc as plsc

N = 10000
E = 320000
D = 128
L = 3

NC = 2
NS = 16
NW = NC * NS
EPW = E // NW
K = 80
NCH = EPW // K
NPAD = 10240
RPT = NPAD // NS

_mesh = plsc.VectorSubcoreMesh(core_axis_name="c", subcore_axis_name="s")


@functools.partial(
    pl.kernel,
    out_type=jax.ShapeDtypeStruct((NC, NPAD), jnp.float32),
    mesh=_mesh,
    scratch_types=[
        pltpu.VMEM((NCH, K), jnp.int32),
        pltpu.VMEM((K,), jnp.float32),
        pltpu.VMEM_SHARED((NPAD,), jnp.float32),
    ],
)
def _deg_kernel(dst_hbm, ones_hbm, zvec_hbm, deg_out, dst_loc, ones_loc, deg_sm):
    c = lax.axis_index("c")
    s = lax.axis_index("s")
    wid = c * NS + s
    pltpu.sync_copy(zvec_hbm, deg_sm.at[pl.ds(s * RPT, RPT)])
    pltpu.sync_copy(dst_hbm.at[wid], dst_loc)
    pltpu.sync_copy(ones_hbm, ones_loc)
    plsc.subcore_barrier()

    def body(j, carry):
        pltpu.sync_copy(ones_loc, deg_sm.at[dst_loc.at[j]], add=True)
        return carry

    lax.fori_loop(0, NCH, body, 0)
    plsc.subcore_barrier()
    pltpu.sync_copy(deg_sm.at[pl.ds(s * RPT, RPT)],
                    deg_out.at[c].at[pl.ds(s * RPT, RPT)])


@functools.partial(
    pl.kernel,
    out_type=jax.ShapeDtypeStruct((NC, NPAD, D), jnp.float32),
    mesh=_mesh,
    scratch_types=[
        pltpu.VMEM((NCH, K), jnp.int32),
        pltpu.VMEM((NCH, K), jnp.int32),
        pltpu.VMEM((K, D), jnp.float32),
        pltpu.VMEM_SHARED((NPAD, D), jnp.float32),
    ],
)
def _prop_kernel(src_hbm, dst_hbm, z_hbm, zrows_hbm, acc_out,
                 src_loc, dst_loc, gbuf, acc_sm):
    c = lax.axis_index("c")
    s = lax.axis_index("s")
    wid = c * NS + s
    pltpu.sync_copy(zrows_hbm, acc_sm.at[pl.ds(s * RPT, RPT)])
    pltpu.sync_copy(src_hbm.at[wid], src_loc)
    pltpu.sync_copy(dst_hbm.at[wid], dst_loc)
    plsc.subcore_barrier()

    def body(j, carry):
        pltpu.sync_copy(z_hbm.at[src_loc.at[j]], gbuf)
        pltpu.sync_copy(gbuf, acc_sm.at[dst_loc.at[j]], add=True)
        return carry

    lax.fori_loop(0, NCH, body, 0)
    plsc.subcore_barrier()
    pltpu.sync_copy(acc_sm.at[pl.ds(s * RPT, RPT)],
                    acc_out.at[c].at[pl.ds(s * RPT, RPT)])


def _dinv_body(deg_ref, o_ref):
    d = deg_ref[0] + deg_ref[1]
    safe = jnp.where(d > 0, d, 1.0)
    o_ref[...] = jnp.where(d > 0, lax.rsqrt(safe), 0.0)


def _prep_body(dinv_ref, x_ref, z_ref):
    z_ref[...] = dinv_ref[...] * x_ref[...]


def _comb_body(acc_ref, dinv_ref, sprev_ref, snew_ref, z_ref):
    sblk = acc_ref[0] + acc_ref[1]
    snew_ref[...] = sprev_ref[...] + sblk
    dv = dinv_ref[...]
    z_ref[...] = dv * dv * sblk


def _final_body(x_ref, dinv_ref, stot_ref, o_ref):
    o_ref[...] = (x_ref[...] + dinv_ref[...] * stot_ref[...]) * 0.25


_RB = 1024
_GRID = NPAD // _RB
_row_spec = pl.BlockSpec((_RB, D), lambda i: (i, 0))
_acc_spec = pl.BlockSpec((NC, _RB, D), lambda i: (0, i, 0))
_sds = lambda: jax.ShapeDtypeStruct((NPAD, D), jnp.float32)

_prep_call = pl.pallas_call(
    _prep_body, grid=(_GRID,), out_shape=_sds(),
    in_specs=[_row_spec, _row_spec], out_specs=_row_spec)

_comb_call = pl.pallas_call(
    _comb_body, grid=(_GRID,), out_shape=(_sds(), _sds()),
    in_specs=[_acc_spec, _row_spec, _row_spec],
    out_specs=(_row_spec, _row_spec))

_final_call = pl.pallas_call(
    _final_body, grid=(_GRID,), out_shape=_sds(),
    in_specs=[_row_spec, _row_spec, _row_spec], out_specs=_row_spec)

_dinv_call = pl.pallas_call(
    _dinv_body, out_shape=jax.ShapeDtypeStruct((NPAD // D, D), jnp.float32))


def kernel(edge_index, emb_weight):
    src_rs = edge_index[0].reshape(NW, NCH, K)
    dst_rs = edge_index[1].reshape(NW, NCH, K)
    ones_k = jnp.ones((K,), jnp.float32)
    zvec = jnp.zeros((NPAD // NS,), jnp.float32)
    zrows = jnp.zeros((NPAD // NS, D), jnp.float32)
    x0 = jnp.concatenate(
        [emb_weight, jnp.zeros((NPAD - N, D), jnp.float32)], axis=0)

    deg2 = _deg_kernel(dst_rs, ones_k, zvec)
    dinv = _dinv_call(deg2.reshape(NC, NPAD // D, D))
    dinvb = jnp.broadcast_to(dinv.reshape(NPAD, 1), (NPAD, D))

    z = _prep_call(dinvb, x0)
    stot = jnp.zeros((NPAD, D), jnp.float32)
    for _ in range(L):
        acc2 = _prop_kernel(src_rs, dst_rs, z, zrows)
        stot, z = _comb_call(acc2, dinvb, stot)
    out = _final_call(x0, dinvb, stot)
    return out[:N]


# speedup vs baseline: 11.8327x; 1.0100x over previous
import functools

import jax
import jax.numpy as jnp
from jax import lax
from jax.experimental import pallas as pl
from jax.experimental.pallas import tpu as pltpu
from jax.experimental.pallas import tpu_sc as plsc

N = 10000
E = 320000
D = 128
L = 3

NC = 2
NS = 16
NW = NC * NS
EPW = E // NW
K = 80
NCH = EPW // K
NPAD = 10240
RPT = NPAD // NS

_mesh = plsc.VectorSubcoreMesh(core_axis_name="c", subcore_axis_name="s")


@functools.partial(
    pl.kernel,
    out_type=jax.ShapeDtypeStruct((NC, NPAD), jnp.float32),
    mesh=_mesh,
    scratch_types=[
        pltpu.VMEM((NCH, K), jnp.int32),
        pltpu.VMEM((K,), jnp.float32),
        pltpu.VMEM_SHARED((NPAD,), jnp.float32),
    ],
)
def _deg_kernel(dst_hbm, ones_hbm, zvec_hbm, deg_out, dst_loc, ones_loc, deg_sm):
    c = lax.axis_index("c")
    s = lax.axis_index("s")
    wid = c * NS + s
    pltpu.sync_copy(zvec_hbm, deg_sm.at[pl.ds(s * RPT, RPT)])
    pltpu.sync_copy(dst_hbm.at[wid], dst_loc)
    pltpu.sync_copy(ones_hbm, ones_loc)
    plsc.subcore_barrier()

    def body(j, carry):
        pltpu.sync_copy(ones_loc, deg_sm.at[dst_loc.at[j]], add=True)
        return carry

    lax.fori_loop(0, NCH, body, 0)
    plsc.subcore_barrier()
    pltpu.sync_copy(deg_sm.at[pl.ds(s * RPT, RPT)],
                    deg_out.at[c].at[pl.ds(s * RPT, RPT)])


@functools.partial(
    pl.kernel,
    out_type=jax.ShapeDtypeStruct((NC, NPAD, D), jnp.float32),
    mesh=_mesh,
    scratch_types=[
        pltpu.VMEM((2, NCH, K), jnp.int32),
        pltpu.VMEM((K, D), jnp.float32),
        pltpu.VMEM_SHARED((NPAD, D), jnp.float32),
    ],
)
def _prop_kernel(edge_hbm, z_hbm, zrows_hbm, acc_out,
                 eidx, gbuf, acc_sm):
    c = lax.axis_index("c")
    s = lax.axis_index("s")
    wid = c * NS + s
    pltpu.sync_copy(zrows_hbm, acc_sm.at[pl.ds(s * RPT, RPT)])
    pltpu.sync_copy(edge_hbm.at[wid], eidx)
    plsc.subcore_barrier()

    def body(j, carry):
        pltpu.sync_copy(z_hbm.at[eidx.at[0, j]], gbuf)
        pltpu.sync_copy(gbuf, acc_sm.at[eidx.at[1, j]], add=True)
        return carry

    lax.fori_loop(0, NCH, body, 0)
    plsc.subcore_barrier()
    pltpu.sync_copy(acc_sm.at[pl.ds(s * RPT, RPT)],
                    acc_out.at[c].at[pl.ds(s * RPT, RPT)])


def _dinv_body(deg_ref, o_ref):
    d = deg_ref[0] + deg_ref[1]
    safe = jnp.where(d > 0, d, 1.0)
    o_ref[...] = jnp.where(d > 0, lax.rsqrt(safe), 0.0)


def _prep_body(dinv_ref, x_ref, z_ref):
    z_ref[...] = dinv_ref[...] * x_ref[...]


def _comb_body(acc_ref, dinv_ref, sprev_ref, snew_ref, z_ref):
    sblk = acc_ref[0] + acc_ref[1]
    snew_ref[...] = sprev_ref[...] + sblk
    dv = dinv_ref[...]
    z_ref[...] = dv * dv * sblk


def _final_body(x_ref, dinv_ref, stot_ref, o_ref):
    o_ref[...] = (x_ref[...] + dinv_ref[...] * stot_ref[...]) * 0.25


_RB = 1024
_GRID = NPAD // _RB
_row_spec = pl.BlockSpec((_RB, D), lambda i: (i, 0))
_acc_spec = pl.BlockSpec((NC, _RB, D), lambda i: (0, i, 0))
_sds = lambda: jax.ShapeDtypeStruct((NPAD, D), jnp.float32)

_prep_call = pl.pallas_call(
    _prep_body, grid=(_GRID,), out_shape=_sds(),
    in_specs=[_row_spec, _row_spec], out_specs=_row_spec)

_comb_call = pl.pallas_call(
    _comb_body, grid=(_GRID,), out_shape=(_sds(), _sds()),
    in_specs=[_acc_spec, _row_spec, _row_spec],
    out_specs=(_row_spec, _row_spec))

_final_call = pl.pallas_call(
    _final_body, grid=(_GRID,), out_shape=_sds(),
    in_specs=[_row_spec, _row_spec, _row_spec], out_specs=_row_spec)

_dinv_call = pl.pallas_call(
    _dinv_body, out_shape=jax.ShapeDtypeStruct((NPAD // D, D), jnp.float32))


def kernel(edge_index, emb_weight):
    src_rs = edge_index[0].reshape(NW, NCH, K)
    dst_rs = edge_index[1].reshape(NW, NCH, K)
    edges_rs = jnp.stack([src_rs, dst_rs], axis=1)
    ones_k = jnp.ones((K,), jnp.float32)
    zvec = jnp.zeros((NPAD // NS,), jnp.float32)
    zrows = jnp.zeros((NPAD // NS, D), jnp.float32)
    x0 = jnp.concatenate(
        [emb_weight, jnp.zeros((NPAD - N, D), jnp.float32)], axis=0)

    deg2 = _deg_kernel(dst_rs, ones_k, zvec)
    dinv = _dinv_call(deg2.reshape(NC, NPAD // D, D))
    dinvb = jnp.broadcast_to(dinv.reshape(NPAD, 1), (NPAD, D))

    z = _prep_call(dinvb, x0)
    stot = jnp.zeros((NPAD, D), jnp.float32)
    for _ in range(L):
        acc2 = _prop_kernel(edges_rs, z, zrows)
        stot, z = _comb_call(acc2, dinvb, stot)
    out = _final_call(x0, dinvb, stot)
    return out[:N]
